# baseline (speedup 1.0000x reference)
import functools

import jax
import jax.numpy as jnp
from jax import lax
from jax.experimental import pallas as pl
from jax.experimental.pallas import tpu as pltpu
from jax.experimental.pallas import tpu_sc as plsc

N_NODES = 10000
DIM = 128
N_EDGES = 320000
N_REL = 964

NC = 2
NS = 16
NW = NC * NS
L = 16

CH = N_EDGES // NW
TPAD = 976
NV = TPAD // L
B = 80
NB = CH // B

_mesh = plsc.VectorSubcoreMesh(core_axis_name="c", subcore_axis_name="s")


def _wid():
    return lax.axis_index("c") * NS + lax.axis_index("s")


@functools.partial(
    pl.kernel,
    out_type=jax.ShapeDtypeStruct((NW, TPAD), jnp.int32),
    mesh=_mesh,
    compiler_params=pltpu.CompilerParams(use_tc_tiling_on_sc=False,
                                         needs_layout_passes=False),
    scratch_types=[
        pltpu.VMEM((CH,), jnp.int32),
        pltpu.VMEM((TPAD,), jnp.int32),
    ],
)
def _hist_kernel(et_hbm, hist_hbm, et_v, h1d):
    wid = _wid()
    pltpu.sync_copy(et_hbm.at[pl.ds(wid * CH, CH)], et_v)

    zero16 = jnp.zeros((L,), jnp.int32)

    def zero_body(j, _):
        h1d[pl.ds(j * L, L)] = zero16
        return 0

    lax.fori_loop(0, NV, zero_body, 0)

    ones = jnp.ones((L,), jnp.int32)

    def hist_body(g, _):
        tv = et_v[pl.ds(g * L, L)]
        plsc.addupdate_scatter(h1d, [tv], ones)
        return 0

    lax.fori_loop(0, CH // L, hist_body, 0)
    pltpu.sync_copy(h1d, hist_hbm.at[wid])


@functools.partial(
    pl.kernel,
    out_type=jax.ShapeDtypeStruct((N_EDGES,), jnp.float32),
    mesh=_mesh,
    compiler_params=pltpu.CompilerParams(use_tc_tiling_on_sc=False,
                                         needs_layout_passes=False),
    scratch_types=[
        pltpu.VMEM((NW, TPAD), jnp.int32),
        pltpu.VMEM((TPAD,), jnp.int32),
        pltpu.VMEM((CH,), jnp.int32),
        pltpu.VMEM((CH,), jnp.int32),
        pltpu.VMEM((CH,), jnp.int32),
        pltpu.VMEM((3, B, DIM), jnp.bfloat16),
        pltpu.VMEM((3, B, DIM), jnp.bfloat16),
        pltpu.VMEM((3, B, DIM), jnp.bfloat16),
        pltpu.VMEM((3, B), jnp.int32),
        pltpu.VMEM((3, B), jnp.float32),
        pltpu.VMEM((B, L), jnp.float32),
        pltpu.VMEM((3 * L,), jnp.int32),
        [pltpu.SemaphoreType.DMA] * 3,
        [pltpu.SemaphoreType.DMA] * 3,
        [pltpu.SemaphoreType.DMA] * 3,
        [pltpu.SemaphoreType.DMA] * 3,
    ],
)
def _main_kernel(x_hbm, left_hbm, right_hbm, et_hbm, r_hbm, hist_hbm, out_hbm,
                 hist_v, base_v, lid_all, rid_all, tid_all, xl2, xr2, rel2,
                 pos2, sc2, part_v, tbuf,
                 glsems, grsems, gtsems, ssems):
    wid = _wid()
    pltpu.sync_copy(hist_hbm, hist_v)
    chunk0 = wid * CH
    pltpu.sync_copy(left_hbm.at[pl.ds(chunk0, CH)], lid_all)
    pltpu.sync_copy(right_hbm.at[pl.ds(chunk0, CH)], rid_all)
    pltpu.sync_copy(et_hbm.at[pl.ds(chunk0, CH)], tid_all)
    wid_v = jnp.zeros((L,), jnp.int32) + wid

    def off_body(j, carry):
        tot = hist_v[0, pl.ds(j * L, L)]
        prior = jnp.where(jnp.zeros((L,), jnp.int32) < wid_v, tot,
                          jnp.zeros((L,), jnp.int32))
        for c in range(1, NW):
            v = hist_v[c, pl.ds(j * L, L)]
            tot = tot + v
            prior = jnp.where(jnp.full((L,), c, jnp.int32) < wid_v,
                              prior + v, prior)
        inc = plsc.cumsum(tot)
        base_v[pl.ds(j * L, L)] = (inc - tot) + prior + carry
        return carry + jnp.sum(tot)

    lax.fori_loop(0, NV, off_body, jnp.int32(0))

    tbuf[pl.ds(0, L)] = jnp.full((L,), -1, jnp.int32)
    tbuf[pl.ds(2 * L, L)] = jnp.full((L,), -2, jnp.int32)
    lanes = lax.iota(jnp.int32, L)
    ones = jnp.ones((L,), jnp.int32)
    zeros = jnp.zeros((L,), jnp.int32)
    fifteen = jnp.full((L,), L - 1, jnp.int32)

    def g_start(b, s):
        i0 = pl.ds(b * B, B)
        pltpu.async_copy(x_hbm.at[lid_all.at[i0]], xl2.at[s], glsems[s])
        pltpu.async_copy(x_hbm.at[rid_all.at[i0]], xr2.at[s], grsems[s])
        pltpu.async_copy(r_hbm.at[tid_all.at[i0]], rel2.at[s], gtsems[s])

    def g_wait(s):
        i0 = pl.ds(0, B)
        pltpu.make_async_copy(x_hbm.at[lid_all.at[i0]], xl2.at[s],
                              glsems[s]).wait()
        pltpu.make_async_copy(x_hbm.at[rid_all.at[i0]], xr2.at[s],
                              grsems[s]).wait()
        pltpu.make_async_copy(r_hbm.at[tid_all.at[i0]], rel2.at[s],
                              gtsems[s]).wait()

    def s_drain(s):
        pltpu.make_async_copy(sc2.at[s], out_hbm.at[pos2.at[s]],
                              ssems[s]).wait()

    def compute(b, s):
        for g in range(B // L):
            tv = tid_all[pl.ds(b * B + g * L, L)]
            tbuf[pl.ds(L, L)] = tv
            rank = zeros
            for k in range(1, L):
                shm = tbuf[pl.ds(L - k, L)]
                rank = rank + jnp.where(shm == tv, ones, zeros)
            gb = plsc.load_gather(base_v, [tv])
            pos2[s, pl.ds(g * L, L)] = gb + rank
            plsc.addupdate_scatter(base_v, [tv], ones)

        @plsc.parallel_loop(0, B, unroll=2)
        def _(i):
            acc = None
            for j in range(DIM // (2 * L)):
                sl = pl.ds(j * 2 * L, 2 * L)
                la, lb = plsc.unpack(xl2[s, i, sl],
                                     format=plsc.PackFormat.INTERLEAVED)
                ra, rb = plsc.unpack(rel2[s, i, sl],
                                     format=plsc.PackFormat.INTERLEAVED)
                xa, xb = plsc.unpack(xr2[s, i, sl],
                                     format=plsc.PackFormat.INTERLEAVED)
                term = la * ra * xa + lb * rb * xb
                acc = term if acc is None else acc + term
            part_v[i] = plsc.cumsum(acc)

        for g in range(B // L):
            eids = lanes + (g * L)
            tot = plsc.load_gather(part_v, [eids, fifteen])
            sc2[s, pl.ds(g * L, L)] = 1.0 / (1.0 + jnp.exp(-tot))

        pltpu.async_copy(sc2.at[s], out_hbm.at[pos2.at[s]], ssems[s])

    g_start(0, 0)
    g_start(1, 1)

    def body(h, _):
        for k in range(3):
            b = 3 * h + k

            @pl.when(h > 0)
            def _():
                s_drain(k)

            g_wait(k)
            g_start(b + 2, (k + 2) % 3)
            compute(b, k)
        return 0

    lax.fori_loop(0, (NB - 2) // 3, body, 0)
    s_drain(0)
    g_wait(0)
    compute(NB - 2, 0)
    s_drain(1)
    g_wait(1)
    compute(NB - 1, 1)
    s_drain(2)
    s_drain(0)
    s_drain(1)


def kernel(x, edge_index, edge_type, R):
    left = edge_index[0]
    right = edge_index[1]
    hist = _hist_kernel(edge_type)
    return _main_kernel(x.astype(jnp.bfloat16), left, right, edge_type,
                        R.astype(jnp.bfloat16), hist)
